# baseline (speedup 1.0000x reference)
import jax
import jax.numpy as jnp
from jax import lax
from jax.experimental import pallas as pl
from jax.experimental.pallas import tpu as pltpu
from jax.experimental.pallas import tpu_sc as plsc

M = 1_000_000
D = 64
NC = 2
NS = 16
NW = NC * NS
W = 768
N_FULL = M // W
TAIL = M - N_FULL * W
TAIL0 = N_FULL * W
ITERS = 42
G = 8
PASSES = W // (16 * G)


def _body(items_hbm, u_hbm, out_hbm, in_buf0, in_buf1, out_buf, u_vmem,
          tail_buf, sem0, sem1, sem_t):
    wid = lax.axis_index("s") * NC + lax.axis_index("c")
    in_bufs = (in_buf0, in_buf1)
    sems = (sem0, sem1)

    pltpu.sync_copy(u_hbm, u_vmem)

    def start_in(j, b):
        col0 = (wid + NW * j) * W
        pltpu.async_copy(items_hbm.at[:, pl.ds(col0, W)], in_bufs[b], sems[b])

    def wait_in(j, b):
        col0 = (wid + NW * j) * W
        pltpu.make_async_copy(items_hbm.at[:, pl.ds(col0, W)], in_bufs[b],
                              sems[b]).wait()

    def compute(j, b):
        buf = in_bufs[b]

        def one_pass(p, _):
            base = p * (16 * G)

            def d_block(db, accs):
                accs = list(accs)
                for k in range(8):
                    d = db * 8 + k
                    u_d = u_vmem[d, :]
                    for g in range(G):
                        v = buf[d, pl.ds(base + g * 16, 16)]
                        accs[g] = accs[g] + v * u_d
                return tuple(accs)

            accs = lax.fori_loop(
                0, 1, d_block,
                tuple(jnp.zeros((16,), jnp.float32) for _ in range(G)))
            for g in range(G):
                out_buf[pl.ds(base + g * 16, 16)] = accs[g]
            return 0

        lax.fori_loop(0, PASSES, one_pass, 0)
        pltpu.sync_copy(out_buf, out_hbm.at[pl.ds((wid + NW * j) * W, W)])

    start_in(0, 0)

    def step(jp, _):
        for b in (0, 1):
            j = 2 * jp + b
            nxt = j + 1
            nxt_valid = jnp.logical_and(nxt < ITERS,
                                        wid + NW * nxt < N_FULL)
            cur_valid = wid + NW * j < N_FULL

            @pl.when(nxt_valid)
            def _():
                start_in(nxt, 1 - b)

            @pl.when(cur_valid)
            def _():
                wait_in(j, b)
                compute(j, b)
        return 0

    lax.fori_loop(0, ITERS // 2, step, 0)

    @pl.when(wid == N_FULL % NW)
    def _():
        pltpu.async_copy(items_hbm.at[:, pl.ds(TAIL0, TAIL)], tail_buf,
                         sem_t).wait()
        def d_block_t(db, accs):
            accs = list(accs)
            for k in range(8):
                d = db * 8 + k
                u_d = u_vmem[d, :]
                for g in range(4):
                    v = tail_buf[d, pl.ds(g * 16, 16)]
                    accs[g] = accs[g] + v * u_d
            return tuple(accs)

        accs = lax.fori_loop(
            0, D // 8, d_block_t,
            tuple(jnp.zeros((16,), jnp.float32) for _ in range(4)))
        for g in range(4):
            out_buf[pl.ds(g * 16, 16)] = accs[g]
        pltpu.sync_copy(out_buf.at[pl.ds(0, TAIL)],
                        out_hbm.at[pl.ds(TAIL0, TAIL)])


@jax.jit
def _sc_matvec(items_t, u_vec):
    mesh = plsc.VectorSubcoreMesh(core_axis_name="c", subcore_axis_name="s")
    f = pl.kernel(
        _body,
        out_type=jax.ShapeDtypeStruct((M,), jnp.float32),
        mesh=mesh,
        scratch_types=[
            pltpu.VMEM((D, W), jnp.float32),
            pltpu.VMEM((D, W), jnp.float32),
            pltpu.VMEM((W,), jnp.float32),
            pltpu.VMEM((D, 16), jnp.float32),
            pltpu.VMEM((D, TAIL), jnp.float32),
            pltpu.SemaphoreType.DMA,
            pltpu.SemaphoreType.DMA,
            pltpu.SemaphoreType.DMA,
        ],
        compiler_params=pltpu.CompilerParams(needs_layout_passes=False,
                                             use_tc_tiling_on_sc=True),
    )
    return f(items_t, u_vec)


def kernel(items_emb, user_emb):
    u_b = jnp.broadcast_to(user_emb.reshape(D, 1), (D, 16))
    return _sc_matvec(items_emb.T, u_b)


# speedup vs baseline: 1.0297x; 1.0297x over previous
import jax
import jax.numpy as jnp
from jax import lax
from jax.experimental import pallas as pl
from jax.experimental.pallas import tpu as pltpu
from jax.experimental.pallas import tpu_sc as plsc

M = 1_000_000
D = 64
NC = 2
NS = 16
NW = NC * NS
W = 768
N_FULL = M // W
TAIL = M - N_FULL * W
TAIL0 = N_FULL * W
ITERS = 42
G = 8
PASSES = W // (16 * G)


def _body(items_hbm, u_hbm, out_hbm, in_buf0, in_buf1, out_buf0, out_buf1,
          u_vmem, tail_buf, sem0, sem1, osem0, osem1, sem_t):
    wid = lax.axis_index("s") * NC + lax.axis_index("c")
    in_bufs = (in_buf0, in_buf1)
    sems = (sem0, sem1)
    out_bufs = (out_buf0, out_buf1)
    osems = (osem0, osem1)

    pltpu.sync_copy(u_hbm, u_vmem)

    def start_in(j, b):
        col0 = (wid + NW * j) * W
        pltpu.async_copy(items_hbm.at[:, pl.ds(col0, W)], in_bufs[b], sems[b])

    def wait_in(j, b):
        col0 = (wid + NW * j) * W
        pltpu.make_async_copy(items_hbm.at[:, pl.ds(col0, W)], in_bufs[b],
                              sems[b]).wait()

    def compute(j, b):
        buf = in_bufs[b]
        out_buf = out_bufs[b]

        def one_pass(p, _):
            base = p * (16 * G)

            def d_block(db, accs):
                accs = list(accs)
                for k in range(8):
                    d = db * 8 + k
                    u_d = u_vmem[d, :]
                    for g in range(G):
                        v = buf[d, pl.ds(base + g * 16, 16)]
                        accs[g] = accs[g] + v * u_d
                return tuple(accs)

            accs = lax.fori_loop(
                0, D // 8, d_block,
                tuple(jnp.zeros((16,), jnp.float32) for _ in range(G)))
            for g in range(G):
                out_buf[pl.ds(base + g * 16, 16)] = accs[g]
            return 0

        lax.fori_loop(0, PASSES, one_pass, 0)
        pltpu.async_copy(out_buf, out_hbm.at[pl.ds((wid + NW * j) * W, W)],
                         osems[b])

    start_in(0, 0)

    def step(jp, _):
        for b in (0, 1):
            j = 2 * jp + b
            nxt = j + 1
            nxt_valid = jnp.logical_and(nxt < ITERS,
                                        wid + NW * nxt < N_FULL)
            cur_valid = wid + NW * j < N_FULL

            @pl.when(nxt_valid)
            def _():
                start_in(nxt, 1 - b)

            @pl.when(cur_valid)
            def _():
                wait_in(j, b)
                prev = j - 2
                @pl.when(prev >= 0)
                def _():
                    pltpu.make_async_copy(
                        out_bufs[b],
                        out_hbm.at[pl.ds((wid + NW * prev) * W, W)],
                        osems[b]).wait()
                compute(j, b)
        return 0

    lax.fori_loop(0, ITERS // 2, step, 0)

    n_valid = (N_FULL - wid + NW - 1) // NW
    for b in (0, 1):
        last_j = ((n_valid - 1 - b) // 2) * 2 + b

        @pl.when(last_j >= 0)
        def _():
            pltpu.make_async_copy(
                out_bufs[b],
                out_hbm.at[pl.ds((wid + NW * last_j) * W, W)],
                osems[b]).wait()

    @pl.when(wid == N_FULL % NW)
    def _():
        pltpu.async_copy(items_hbm.at[:, pl.ds(TAIL0, TAIL)], tail_buf,
                         sem_t).wait()
        def d_block_t(db, accs):
            accs = list(accs)
            for k in range(8):
                d = db * 8 + k
                u_d = u_vmem[d, :]
                for g in range(4):
                    v = tail_buf[d, pl.ds(g * 16, 16)]
                    accs[g] = accs[g] + v * u_d
            return tuple(accs)

        accs = lax.fori_loop(
            0, D // 8, d_block_t,
            tuple(jnp.zeros((16,), jnp.float32) for _ in range(4)))
        for g in range(4):
            out_buf0[pl.ds(g * 16, 16)] = accs[g]
        pltpu.sync_copy(out_buf0.at[pl.ds(0, TAIL)],
                        out_hbm.at[pl.ds(TAIL0, TAIL)])


@jax.jit
def _sc_matvec(items_t, u_vec):
    mesh = plsc.VectorSubcoreMesh(core_axis_name="c", subcore_axis_name="s")
    f = pl.kernel(
        _body,
        out_type=jax.ShapeDtypeStruct((M,), jnp.float32),
        mesh=mesh,
        scratch_types=[
            pltpu.VMEM((D, W), jnp.float32),
            pltpu.VMEM((D, W), jnp.float32),
            pltpu.VMEM((W,), jnp.float32),
            pltpu.VMEM((W,), jnp.float32),
            pltpu.VMEM((D, 16), jnp.float32),
            pltpu.VMEM((D, TAIL), jnp.float32),
            pltpu.SemaphoreType.DMA,
            pltpu.SemaphoreType.DMA,
            pltpu.SemaphoreType.DMA,
            pltpu.SemaphoreType.DMA,
            pltpu.SemaphoreType.DMA,
        ],
        compiler_params=pltpu.CompilerParams(needs_layout_passes=False,
                                             use_tc_tiling_on_sc=True),
    )
    return f(items_t, u_vec)


def kernel(items_emb, user_emb):
    u_b = jnp.broadcast_to(user_emb.reshape(D, 1), (D, 16))
    return _sc_matvec(items_emb.T, u_b)
